# baseline (speedup 1.0000x reference)
import functools

import jax
import jax.numpy as jnp
from jax.experimental import pallas as pl
from jax.experimental.pallas import tpu as pltpu

_N = 2_000_000
_COLS = 128
_ROWS = _N // _COLS
_BLOCK_ROWS = 2000
_GRID = -(-_ROWS // _BLOCK_ROWS)


def _tc_body(s_ref, t_ref, x_ref, o_ref):
    o_ref[...] = x_ref[...] * s_ref[0, 0] + t_ref[0, 0]


@functools.partial(jax.jit, static_argnames=())
def _tc_affine(x2, s11, t11):
    return pl.pallas_call(
        _tc_body,
        grid=(_GRID,),
        in_specs=[
            pl.BlockSpec(memory_space=pltpu.SMEM),
            pl.BlockSpec(memory_space=pltpu.SMEM),
            pl.BlockSpec((_BLOCK_ROWS, _COLS), lambda i: (i, 0)),
        ],
        out_specs=pl.BlockSpec((_BLOCK_ROWS, _COLS), lambda i: (i, 0)),
        out_shape=jax.ShapeDtypeStruct((_ROWS, _COLS), jnp.float32),
    )(s11, t11, x2)


def kernel(x, head, scale, shift):
    del head
    x2 = jnp.reshape(x, (_ROWS, _COLS))
    s11 = jnp.reshape(scale.astype(jnp.float32), (1, 1))
    t11 = jnp.reshape(shift.astype(jnp.float32), (1, 1))
    return jnp.reshape(_tc_affine(x2, s11, t11), (_N,))


# speedup vs baseline: 1.4650x; 1.4650x over previous
import functools

import jax
import jax.numpy as jnp
from jax.experimental import pallas as pl
from jax.experimental.pallas import tpu as pltpu

_N = 2_000_000
_COLS = 128
_ROWS = _N // _COLS
_BLOCK_ROWS = 8000
_GRID = -(-_ROWS // _BLOCK_ROWS)


def _tc_body(s_ref, t_ref, x_ref, o_ref):
    o_ref[...] = x_ref[...] * s_ref[0, 0] + t_ref[0, 0]


@functools.partial(jax.jit, static_argnames=())
def _tc_affine(x2, s11, t11):
    return pl.pallas_call(
        _tc_body,
        grid=(_GRID,),
        in_specs=[
            pl.BlockSpec(memory_space=pltpu.SMEM),
            pl.BlockSpec(memory_space=pltpu.SMEM),
            pl.BlockSpec((_BLOCK_ROWS, _COLS), lambda i: (i, 0)),
        ],
        out_specs=pl.BlockSpec((_BLOCK_ROWS, _COLS), lambda i: (i, 0)),
        out_shape=jax.ShapeDtypeStruct((_ROWS, _COLS), jnp.float32),
    )(s11, t11, x2)


def kernel(x, head, scale, shift):
    del head
    x2 = jnp.reshape(x, (_ROWS, _COLS))
    s11 = jnp.reshape(scale.astype(jnp.float32), (1, 1))
    t11 = jnp.reshape(shift.astype(jnp.float32), (1, 1))
    return jnp.reshape(_tc_affine(x2, s11, t11), (_N,))
